# baseline (speedup 1.0000x reference)
import functools

import jax
import jax.numpy as jnp
from jax import lax
from jax.experimental import pallas as pl
from jax.experimental.pallas import tpu as pltpu
from jax.experimental.pallas import tpu_sc as plsc

_NC = 2
_NS = 16
_CHUNK = 64
_G = 32


def _stage1_body(up_ref, x_ref, wtop_ref, wbot_ref, bmu_ref, uw_ref, xw_ref):
    uw_ref[...] = (
        jnp.dot(up_ref[...], wbot_ref[...], preferred_element_type=jnp.float32)
        + bmu_ref[...]
    )

    @pl.when(pl.program_id(0) == 0)
    def _():
        xw_ref[...] = jnp.dot(
            x_ref[...], wtop_ref[...], preferred_element_type=jnp.float32
        )


def _stage1(up_attr, x, wtop, wbot, bmu2):
    E, D = up_attr.shape
    N = x.shape[0]
    BE = next(b for b in (8192, 4096, 2048, 1024, 512, 256, 128)
              if E % b == 0)
    grid = E // BE
    return pl.pallas_call(
        _stage1_body,
        grid=(grid,),
        in_specs=[
            pl.BlockSpec((BE, D), lambda i: (i, 0)),
            pl.BlockSpec((N, D), lambda i: (0, 0)),
            pl.BlockSpec((D, D), lambda i: (0, 0)),
            pl.BlockSpec((D, D), lambda i: (0, 0)),
            pl.BlockSpec((1, D), lambda i: (0, 0)),
        ],
        out_specs=[
            pl.BlockSpec((BE, D), lambda i: (i, 0)),
            pl.BlockSpec((N, D), lambda i: (0, 0)),
        ],
        out_shape=[
            jax.ShapeDtypeStruct((E, D), jnp.float32),
            jax.ShapeDtypeStruct((N, D), jnp.float32),
        ],
    )(up_attr, x, wtop, wbot, bmu2)


def _sc_segment_body(NP, E, EBP, D,
                     xw_hbm, uw_hbm, battr_hbm, src_hbm, dst_hbm,
                     bsrc_hbm, bdst_hbm,
                     up_parts_hbm, b_parts_hbm,
                     idx_s, idx_d, uw_a, uw_b, xg_a, xg_b, acc, sem_a, sem_b):
    c = lax.axis_index("c")
    s = lax.axis_index("s")
    gw = s * _NC + c
    nw = _NC * _NS
    rps = NP // _NS
    NCH = E // _CHUNK
    NBCH = EBP // _CHUNK
    nslc = D // 16
    MAXCH = NCH // nw
    MAXB = NBCH // nw

    def _zero_buf(buf):
        z = jnp.zeros((16,), jnp.float32)

        @plsc.parallel_loop(0, _CHUNK)
        def _(r):
            for j in range(nslc):
                buf[r, pl.ds(j * 16, 16)] = z

    def _zero_acc():
        base = s * rps
        nfull = rps // _CHUNK
        rem = rps - nfull * _CHUNK

        def _z(i, _):
            pltpu.sync_copy(xg_a, acc.at[pl.ds(base + i * _CHUNK, _CHUNK)])
            return 0

        lax.fori_loop(0, nfull, _z, 0)
        if rem:
            pltpu.sync_copy(
                xg_a.at[pl.ds(0, rem)],
                acc.at[pl.ds(base + nfull * _CHUNK, rem)],
            )

    def _relu_add(uw_v, xg_v):
        @plsc.parallel_loop(0, _CHUNK)
        def _(r):
            for j in range(nslc):
                sl = pl.ds(j * 16, 16)
                uw_v[r, sl] = jnp.maximum(uw_v[r, sl] + xg_v[r, sl], 0.0)

    def _issue(rel, grow, xg_v, uw_v, sem):
        pltpu.async_copy(xw_hbm.at[idx_s.at[rel]], xg_v, sem)
        pltpu.async_copy(uw_hbm.at[pl.ds((grow + rel) * _CHUNK, _CHUNK)],
                         uw_v, sem)

    def _drain2(sem, buf):
        pltpu.make_async_copy(uw_hbm.at[pl.ds(0, _CHUNK)], buf, sem).wait()
        pltpu.make_async_copy(uw_hbm.at[pl.ds(0, _CHUNK)], buf, sem).wait()

    lo = gw * MAXCH

    _zero_buf(xg_a)
    _zero_acc()
    plsc.subcore_barrier()

    def _group(g, _):
        grow = lo + g * _G
        pltpu.sync_copy(src_hbm.at[pl.ds(grow, _G)], idx_s)
        pltpu.sync_copy(dst_hbm.at[pl.ds(grow, _G)], idx_d)

        _issue(0, grow, xg_a, uw_a, sem_a)
        _issue(1, grow, xg_b, uw_b, sem_b)
        for k in range(_G // 2):
            ra = 2 * k
            _drain2(sem_a, xg_a)
            _relu_add(uw_a, xg_a)
            pltpu.sync_copy(uw_a, acc.at[idx_d.at[ra]], add=True)
            if ra + 2 < _G:
                _issue(ra + 2, grow, xg_a, uw_a, sem_a)
            rb = ra + 1
            _drain2(sem_b, xg_b)
            _relu_add(uw_b, xg_b)
            pltpu.sync_copy(uw_b, acc.at[idx_d.at[rb]], add=True)
            if rb + 2 < _G:
                _issue(rb + 2, grow, xg_b, uw_b, sem_b)
        return 0

    lax.fori_loop(0, MAXCH // _G, _group, 0)
    plsc.subcore_barrier()

    row0 = s * rps
    pltpu.sync_copy(
        acc.at[pl.ds(row0, rps)],
        up_parts_hbm.at[c, pl.ds(row0, rps)],
    )
    plsc.subcore_barrier()

    lob = gw * MAXB

    _zero_buf(xg_a)
    _zero_acc()
    plsc.subcore_barrier()

    def _bgroup(g, _):
        grow = lob + g * _G
        pltpu.sync_copy(bsrc_hbm.at[pl.ds(grow, _G)], idx_s)
        pltpu.sync_copy(bdst_hbm.at[pl.ds(grow, _G)], idx_d)

        def _b_chunk(i, _2):
            pltpu.async_copy(battr_hbm.at[idx_s.at[i]], xg_b, sem_b).wait()
            pltpu.sync_copy(xg_b, acc.at[idx_d.at[i]], add=True)
            return 0

        lax.fori_loop(0, _G, _b_chunk, 0)
        return 0

    lax.fori_loop(0, MAXB // _G, _bgroup, 0)
    plsc.subcore_barrier()

    pltpu.sync_copy(
        acc.at[pl.ds(row0, rps)],
        b_parts_hbm.at[c, pl.ds(row0, rps)],
    )


def _sc_segment(xw, uw, battr, src2, dst2, bsrc2, bdst2, NP):
    E = src2.shape[0] * src2.shape[1]
    EBP = bsrc2.shape[0] * bsrc2.shape[1]
    D = xw.shape[1]
    mesh = plsc.VectorSubcoreMesh(
        core_axis_name="c", subcore_axis_name="s",
        num_cores=_NC, num_subcores=_NS,
    )
    fn = pl.kernel(
        functools.partial(_sc_segment_body, NP, E, EBP, D),
        out_type=[
            jax.ShapeDtypeStruct((_NC, NP, D), jnp.float32),
            jax.ShapeDtypeStruct((_NC, NP, D), jnp.float32),
        ],
        mesh=mesh,
        scratch_types=[
            pltpu.VMEM((_G, _CHUNK), jnp.int32),
            pltpu.VMEM((_G, _CHUNK), jnp.int32),
            pltpu.VMEM((_CHUNK, D), jnp.float32),
            pltpu.VMEM((_CHUNK, D), jnp.float32),
            pltpu.VMEM((_CHUNK, D), jnp.float32),
            pltpu.VMEM((_CHUNK, D), jnp.float32),
            pltpu.VMEM_SHARED((NP, D), jnp.float32),
            pltpu.SemaphoreType.DMA,
            pltpu.SemaphoreType.DMA,
        ],
    )
    return fn(xw, uw, battr, src2, dst2, bsrc2, bdst2)


def _stage2_body(N,
                 up_parts, b_parts, x_ref,
                 wu1, bu1, gu1, beu1, wu2, bu2, gu2, beu2,
                 wb1, bb1, gb1, beb1, wb2, bb2, gb2, beb2,
                 wc1, wc2, bc, gc, bec, out_ref):
    def bn_relu(t, g, b):
        mu = jnp.mean(t, axis=0, keepdims=True)
        var = jnp.mean((t - mu) ** 2, axis=0, keepdims=True)
        return jnp.maximum(g * (t - mu) / jnp.sqrt(var + 1e-5) + b, 0.0)

    xv = x_ref[...]
    ou = up_parts[0, :N, :] + up_parts[1, :N, :] + xv
    ob = b_parts[0, :N, :] + b_parts[1, :N, :] + xv

    h1 = bn_relu(jnp.dot(ou, wu1[...], preferred_element_type=jnp.float32)
                 + bu1[...], gu1[...], beu1[...])
    h1 = bn_relu(jnp.dot(h1, wu2[...], preferred_element_type=jnp.float32)
                 + bu2[...], gu2[...], beu2[...])
    h2 = bn_relu(jnp.dot(ob, wb1[...], preferred_element_type=jnp.float32)
                 + bb1[...], gb1[...], beb1[...])
    h2 = bn_relu(jnp.dot(h2, wb2[...], preferred_element_type=jnp.float32)
                 + bb2[...], gb2[...], beb2[...])
    comb = (jnp.dot(h1, wc1[...], preferred_element_type=jnp.float32)
            + jnp.dot(h2, wc2[...], preferred_element_type=jnp.float32)
            + bc[...])
    out_ref[...] = bn_relu(comb, gc[...], bec[...])


def _stage2(up_parts, b_parts, x, *weights):
    N, D = x.shape
    H = weights[0].shape[1]
    return pl.pallas_call(
        functools.partial(_stage2_body, N),
        out_shape=jax.ShapeDtypeStruct((N, H), jnp.float32),
    )(up_parts, b_parts, x, *weights)


def kernel(x, up_attr, boundary_attr, Wmu, bmu, Wu1, bu1, gu1, beu1, Wu2, bu2,
           gu2, beu2, Wb1, bb1, gb1, beb1, Wb2, bb2, gb2, beb2, Wc, bc, gc,
           bec, up_index, boundary_index):
    N, D = x.shape
    E = up_index.shape[1]
    EB = boundary_index.shape[1]
    H = Wu1.shape[1]

    NP = ((N + 1 + _NS * 8 - 1) // (_NS * 8)) * (_NS * 8)

    wtop = Wmu[:D]
    wbot = Wmu[D:]
    bmu2 = bmu.reshape(1, D)

    nw = _NC * _NS

    def _pad_edges(srcv, dstv, ecount):
        nch = (ecount + _CHUNK - 1) // _CHUNK
        maxch = (((nch + nw - 1) // nw + _G - 1) // _G) * _G
        ep = nw * maxch * _CHUNK
        pad = ep - ecount
        if pad:
            srcv = jnp.concatenate([srcv, jnp.zeros((pad,), jnp.int32)])
            dstv = jnp.concatenate([dstv, jnp.full((pad,), N, jnp.int32)])
        return (srcv.reshape(ep // _CHUNK, _CHUNK),
                dstv.reshape(ep // _CHUNK, _CHUNK), ep)

    src2, dst2, EP = _pad_edges(up_index[0], up_index[1], E)
    bsrc2, bdst2, EBP = _pad_edges(boundary_index[0], boundary_index[1], EB)

    if EP > E:
        up_attr = jnp.concatenate(
            [up_attr, jnp.zeros((EP - E, D), jnp.float32)])

    uw, xw = _stage1(up_attr, x, wtop, wbot, bmu2)

    up_parts, b_parts = _sc_segment(xw, uw, boundary_attr, src2, dst2, bsrc2,
                                    bdst2, NP)

    out = _stage2(
        up_parts, b_parts, x,
        Wu1, bu1.reshape(1, H), gu1.reshape(1, H), beu1.reshape(1, H),
        Wu2, bu2.reshape(1, H), gu2.reshape(1, H), beu2.reshape(1, H),
        Wb1, bb1.reshape(1, H), gb1.reshape(1, H), beb1.reshape(1, H),
        Wb2, bb2.reshape(1, H), gb2.reshape(1, H), beb2.reshape(1, H),
        Wc[:H], Wc[H:], bc.reshape(1, H), gc.reshape(1, H), bec.reshape(1, H),
    )
    return out


# speedup vs baseline: 2.1090x; 2.1090x over previous
import functools

import jax
import jax.numpy as jnp
from jax import lax
from jax.experimental import pallas as pl
from jax.experimental.pallas import tpu as pltpu
from jax.experimental.pallas import tpu_sc as plsc

_NC = 2
_NS = 16
_CHUNK = 64
_G = 16


def _stage1_body(up_ref, x_ref, wtop_ref, wbot_ref, bmu_ref, uw_ref, xw_ref):
    uw_ref[...] = (
        jnp.dot(up_ref[...], wbot_ref[...], preferred_element_type=jnp.float32)
        + bmu_ref[...]
    )

    @pl.when(pl.program_id(0) == 0)
    def _():
        xw_ref[...] = jnp.dot(
            x_ref[...], wtop_ref[...], preferred_element_type=jnp.float32
        )


def _stage1(up_attr, x, wtop, wbot, bmu2):
    E, D = up_attr.shape
    N = x.shape[0]
    BE = next(b for b in (8192, 4096, 2048, 1024, 512, 256, 128)
              if E % b == 0)
    grid = E // BE
    return pl.pallas_call(
        _stage1_body,
        grid=(grid,),
        in_specs=[
            pl.BlockSpec((BE, D), lambda i: (i, 0)),
            pl.BlockSpec((N, D), lambda i: (0, 0)),
            pl.BlockSpec((D, D), lambda i: (0, 0)),
            pl.BlockSpec((D, D), lambda i: (0, 0)),
            pl.BlockSpec((1, D), lambda i: (0, 0)),
        ],
        out_specs=[
            pl.BlockSpec((BE, D), lambda i: (i, 0)),
            pl.BlockSpec((N, D), lambda i: (0, 0)),
        ],
        out_shape=[
            jax.ShapeDtypeStruct((E, D), jnp.float32),
            jax.ShapeDtypeStruct((N, D), jnp.float32),
        ],
    )(up_attr, x, wtop, wbot, bmu2)


def _sc_segment_body(NP, E, EBP, D,
                     xw_hbm, uw_hbm, battr_hbm, src_hbm, dst_hbm,
                     bsrc_hbm, bdst_hbm,
                     up_parts_hbm, b_parts_hbm,
                     idx_s, idx_d, uw_a, uw_b, xg_a, xg_b, acc, sem_a, sem_b):
    c = lax.axis_index("c")
    s = lax.axis_index("s")
    gw = s * _NC + c
    nw = _NC * _NS
    rps = NP // _NS
    NCH = E // _CHUNK
    NBCH = EBP // _CHUNK
    nslc = D // 16
    MAXCH = NCH // nw
    MAXB = NBCH // nw

    def _zero_buf(buf):
        z = jnp.zeros((16,), jnp.float32)

        @plsc.parallel_loop(0, _CHUNK)
        def _(r):
            for j in range(nslc):
                buf[r, pl.ds(j * 16, 16)] = z

    def _zero_acc():
        base = s * rps
        nfull = rps // _CHUNK
        rem = rps - nfull * _CHUNK

        def _z(i, _):
            pltpu.sync_copy(xg_a, acc.at[pl.ds(base + i * _CHUNK, _CHUNK)])
            return 0

        lax.fori_loop(0, nfull, _z, 0)
        if rem:
            pltpu.sync_copy(
                xg_a.at[pl.ds(0, rem)],
                acc.at[pl.ds(base + nfull * _CHUNK, rem)],
            )

    def _relu_add(uw_v, xg_v):
        @plsc.parallel_loop(0, _CHUNK)
        def _(r):
            for j in range(nslc):
                sl = pl.ds(j * 16, 16)
                uw_v[r, sl] = jnp.maximum(uw_v[r, sl] + xg_v[r, sl], 0.0)

    def _issue(rel, grow, xg_v, uw_v, sem):
        pltpu.async_copy(xw_hbm.at[idx_s.at[rel]], xg_v, sem)
        pltpu.async_copy(uw_hbm.at[pl.ds((grow + rel) * _CHUNK, _CHUNK)],
                         uw_v, sem)

    def _drain2(sem, buf):
        pltpu.make_async_copy(uw_hbm.at[pl.ds(0, _CHUNK)], buf, sem).wait()
        pltpu.make_async_copy(uw_hbm.at[pl.ds(0, _CHUNK)], buf, sem).wait()

    lo = gw * MAXCH

    _zero_buf(xg_a)
    _zero_acc()
    plsc.subcore_barrier()

    def _group(g, _):
        grow = lo + g * _G
        pltpu.sync_copy(src_hbm.at[pl.ds(grow, _G)], idx_s)
        pltpu.sync_copy(dst_hbm.at[pl.ds(grow, _G)], idx_d)

        _issue(0, grow, xg_a, uw_a, sem_a)
        _issue(1, grow, xg_b, uw_b, sem_b)
        for k in range(_G // 2):
            ra = 2 * k
            _drain2(sem_a, xg_a)
            _relu_add(uw_a, xg_a)
            pltpu.sync_copy(uw_a, acc.at[idx_d.at[ra]], add=True)
            if ra + 2 < _G:
                _issue(ra + 2, grow, xg_a, uw_a, sem_a)
            rb = ra + 1
            _drain2(sem_b, xg_b)
            _relu_add(uw_b, xg_b)
            pltpu.sync_copy(uw_b, acc.at[idx_d.at[rb]], add=True)
            if rb + 2 < _G:
                _issue(rb + 2, grow, xg_b, uw_b, sem_b)
        return 0

    lax.fori_loop(0, MAXCH // _G, _group, 0)
    plsc.subcore_barrier()

    row0 = s * rps
    pltpu.sync_copy(
        acc.at[pl.ds(row0, rps)],
        up_parts_hbm.at[c, pl.ds(row0, rps)],
    )
    plsc.subcore_barrier()

    lob = gw * MAXB

    _zero_buf(xg_a)
    _zero_acc()
    plsc.subcore_barrier()

    def _bgroup(g, _):
        grow = lob + g * _G
        pltpu.sync_copy(bsrc_hbm.at[pl.ds(grow, _G)], idx_s)
        pltpu.sync_copy(bdst_hbm.at[pl.ds(grow, _G)], idx_d)

        def _b_chunk(i, _2):
            pltpu.async_copy(battr_hbm.at[idx_s.at[i]], xg_b, sem_b).wait()
            pltpu.sync_copy(xg_b, acc.at[idx_d.at[i]], add=True)
            return 0

        lax.fori_loop(0, _G, _b_chunk, 0)
        return 0

    lax.fori_loop(0, MAXB // _G, _bgroup, 0)
    plsc.subcore_barrier()

    pltpu.sync_copy(
        acc.at[pl.ds(row0, rps)],
        b_parts_hbm.at[c, pl.ds(row0, rps)],
    )


def _sc_segment(xw, uw, battr, src2, dst2, bsrc2, bdst2, NP):
    E = src2.shape[0] * src2.shape[1]
    EBP = bsrc2.shape[0] * bsrc2.shape[1]
    D = xw.shape[1]
    mesh = plsc.VectorSubcoreMesh(
        core_axis_name="c", subcore_axis_name="s",
        num_cores=_NC, num_subcores=_NS,
    )
    fn = pl.kernel(
        functools.partial(_sc_segment_body, NP, E, EBP, D),
        out_type=[
            jax.ShapeDtypeStruct((_NC, NP, D), jnp.float32),
            jax.ShapeDtypeStruct((_NC, NP, D), jnp.float32),
        ],
        mesh=mesh,
        scratch_types=[
            pltpu.VMEM((_G, _CHUNK), jnp.int32),
            pltpu.VMEM((_G, _CHUNK), jnp.int32),
            pltpu.VMEM((_CHUNK, D), jnp.float32),
            pltpu.VMEM((_CHUNK, D), jnp.float32),
            pltpu.VMEM((_CHUNK, D), jnp.float32),
            pltpu.VMEM((_CHUNK, D), jnp.float32),
            pltpu.VMEM_SHARED((NP, D), jnp.float32),
            pltpu.SemaphoreType.DMA,
            pltpu.SemaphoreType.DMA,
        ],
    )
    return fn(xw, uw, battr, src2, dst2, bsrc2, bdst2)


def _stage2_body(N,
                 up_parts, b_parts, x_ref,
                 wu1, bu1, gu1, beu1, wu2, bu2, gu2, beu2,
                 wb1, bb1, gb1, beb1, wb2, bb2, gb2, beb2,
                 wc1, wc2, bc, gc, bec, out_ref):
    def bn_relu(t, g, b):
        mu = jnp.mean(t, axis=0, keepdims=True)
        var = jnp.mean((t - mu) ** 2, axis=0, keepdims=True)
        return jnp.maximum(g * (t - mu) / jnp.sqrt(var + 1e-5) + b, 0.0)

    xv = x_ref[...]
    ou = up_parts[0, :N, :] + up_parts[1, :N, :] + xv
    ob = b_parts[0, :N, :] + b_parts[1, :N, :] + xv

    h1 = bn_relu(jnp.dot(ou, wu1[...], preferred_element_type=jnp.float32)
                 + bu1[...], gu1[...], beu1[...])
    h1 = bn_relu(jnp.dot(h1, wu2[...], preferred_element_type=jnp.float32)
                 + bu2[...], gu2[...], beu2[...])
    h2 = bn_relu(jnp.dot(ob, wb1[...], preferred_element_type=jnp.float32)
                 + bb1[...], gb1[...], beb1[...])
    h2 = bn_relu(jnp.dot(h2, wb2[...], preferred_element_type=jnp.float32)
                 + bb2[...], gb2[...], beb2[...])
    comb = (jnp.dot(h1, wc1[...], preferred_element_type=jnp.float32)
            + jnp.dot(h2, wc2[...], preferred_element_type=jnp.float32)
            + bc[...])
    out_ref[...] = bn_relu(comb, gc[...], bec[...])


def _stage2(up_parts, b_parts, x, *weights):
    N, D = x.shape
    H = weights[0].shape[1]
    return pl.pallas_call(
        functools.partial(_stage2_body, N),
        out_shape=jax.ShapeDtypeStruct((N, H), jnp.float32),
    )(up_parts, b_parts, x, *weights)


def kernel(x, up_attr, boundary_attr, Wmu, bmu, Wu1, bu1, gu1, beu1, Wu2, bu2,
           gu2, beu2, Wb1, bb1, gb1, beb1, Wb2, bb2, gb2, beb2, Wc, bc, gc,
           bec, up_index, boundary_index):
    N, D = x.shape
    E = up_index.shape[1]
    EB = boundary_index.shape[1]
    H = Wu1.shape[1]

    NP = ((N + 1 + _NS * 8 - 1) // (_NS * 8)) * (_NS * 8)

    wtop = Wmu[:D]
    wbot = Wmu[D:]
    bmu2 = bmu.reshape(1, D)

    nw = _NC * _NS

    def _pad_edges(srcv, dstv, ecount):
        nch = (ecount + _CHUNK - 1) // _CHUNK
        maxch = (((nch + nw - 1) // nw + _G - 1) // _G) * _G
        ep = nw * maxch * _CHUNK
        pad = ep - ecount
        if pad:
            srcv = jnp.concatenate([srcv, jnp.zeros((pad,), jnp.int32)])
            dstv = jnp.concatenate([dstv, jnp.full((pad,), N, jnp.int32)])
        return (srcv.reshape(ep // _CHUNK, _CHUNK),
                dstv.reshape(ep // _CHUNK, _CHUNK), ep)

    src2, dst2, EP = _pad_edges(up_index[0], up_index[1], E)
    bsrc2, bdst2, EBP = _pad_edges(boundary_index[0], boundary_index[1], EB)

    if EP > E:
        up_attr = jnp.concatenate(
            [up_attr, jnp.zeros((EP - E, D), jnp.float32)])

    uw, xw = _stage1(up_attr, x, wtop, wbot, bmu2)

    up_parts, b_parts = _sc_segment(xw, uw, boundary_attr, src2, dst2, bsrc2,
                                    bdst2, NP)

    out = _stage2(
        up_parts, b_parts, x,
        Wu1, bu1.reshape(1, H), gu1.reshape(1, H), beu1.reshape(1, H),
        Wu2, bu2.reshape(1, H), gu2.reshape(1, H), beu2.reshape(1, H),
        Wb1, bb1.reshape(1, H), gb1.reshape(1, H), beb1.reshape(1, H),
        Wb2, bb2.reshape(1, H), gb2.reshape(1, H), beb2.reshape(1, H),
        Wc[:H], Wc[H:], bc.reshape(1, H), gc.reshape(1, H), bec.reshape(1, H),
    )
    return out
